# baseline (speedup 1.0000x reference)
import functools

import jax
import jax.numpy as jnp
from jax import lax
from jax.experimental import pallas as pl
from jax.experimental.pallas import tpu as pltpu
from jax.experimental.pallas import tpu_sc as plsc

_L = 16


@functools.lru_cache(maxsize=None)
def _make_lookup(V, D, B0, B1):
    info = plsc.get_sparse_core_info()
    NC, NS = info.num_cores, info.num_subcores
    NW = NC * NS
    assert D == 32 and B0 % (NW * 128) == 0
    W = B0 // NW
    TC_W = W // 128
    mesh = plsc.VectorSubcoreMesh(core_axis_name="c", subcore_axis_name="s")

    @functools.partial(
        pl.kernel,
        mesh=mesh,
        out_type=jax.ShapeDtypeStruct((B1, D // 8, B0 // 128, 8, 128),
                                      jnp.float32),
        scratch_types=[
            pltpu.VMEM((B1, W), jnp.int32),
            pltpu.VMEM((W, D), jnp.float32),
            pltpu.VMEM((W, D), jnp.float32),
            pltpu.VMEM((D // 8, TC_W, 10, 129), jnp.float32),
            pltpu.VMEM((D // 8, TC_W, 10, 129), jnp.float32),
            pltpu.SemaphoreType.DMA,
            pltpu.SemaphoreType.DMA,
            pltpu.SemaphoreType.DMA,
            pltpu.SemaphoreType.DMA,
        ],
        compiler_params=pltpu.CompilerParams(
            use_tc_tiling_on_sc=False, needs_layout_passes=False),
    )
    def k(tbl_hbm, xt_hbm, out_hbm, xb, rows0, rows1, ob0, ob1,
          sg0, sg1, so0, so1):
        wid = lax.axis_index("s") * NC + lax.axis_index("c")
        b0_base = wid * W
        tc0 = wid * TC_W
        rows = (rows0, rows1)
        ob = (ob0, ob1)
        sg = (sg0, sg1)
        so = (so0, so1)
        lane = lax.iota(jnp.int32, _L)

        pltpu.sync_copy(xt_hbm.at[:, pl.ds(b0_base, W)], xb)

        def gather_copy(c, buf):
            return pltpu.make_async_copy(
                tbl_hbm.at[xb.at[c]], rows[buf], sg[buf])

        def body(c, buf):
            @pl.when(c + 1 < B1)
            def _():
                gather_copy(c + 1, 1 - buf).start()
            gather_copy(c, buf).wait()
            @pl.when(c >= 2)
            def _():
                pltpu.make_async_copy(
                    ob[buf].at[:, :, pl.ds(0, 8), pl.ds(0, 128)],
                    out_hbm.at[0, :, pl.ds(0, TC_W)],
                    so[buf]).wait()
            trv = (lane // 8, 2 + lane // 8)
            sv = lax.rem(lane, 8)

            @pl.loop(0, W, unroll=8)
            def _(j):
                jcv = jnp.full((_L,), 0, jnp.int32) + (j // 128)
                lv = jnp.full((_L,), 0, jnp.int32) + lax.rem(j, 128)
                for kk in range(2):
                    vals = rows[buf][j, pl.ds(kk * _L, _L)]
                    plsc.store_scatter(
                        ob[buf], [trv[kk], jcv, sv, lv], vals)

            pltpu.async_copy(
                ob[buf].at[:, :, pl.ds(0, 8), pl.ds(0, 128)],
                out_hbm.at[c, :, pl.ds(tc0, TC_W)], so[buf])

        gather_copy(0, 0).start()

        @pl.loop(0, B1)
        def _(c):
            buf = lax.rem(c, 2)

            @pl.when(buf == 0)
            def _():
                body(c, 0)

            @pl.when(buf == 1)
            def _():
                body(c, 1)

        pltpu.make_async_copy(
            ob0.at[:, :, pl.ds(0, 8), pl.ds(0, 128)],
            out_hbm.at[0, :, pl.ds(0, TC_W)], so0).wait()
        pltpu.make_async_copy(
            ob1.at[:, :, pl.ds(0, 8), pl.ds(0, 128)],
            out_hbm.at[0, :, pl.ds(0, TC_W)], so1).wait()

    return k


def kernel(x, embeddings):
    V, D = embeddings.shape
    B0, B1 = x.shape
    xt = x.T.astype(jnp.int32)
    out5 = _make_lookup(V, D, B0, B1)(embeddings, xt)
    out = out5.transpose(0, 1, 3, 2, 4).reshape(B1, D, B0)
    return out.transpose(2, 0, 1)
